# baseline (speedup 1.0000x reference)
import functools

import jax
import jax.numpy as jnp
from jax import lax
from jax.experimental import pallas as pl
from jax.experimental.pallas import tpu as pltpu
from jax.experimental.pallas import tpu_sc as plsc

_N = 100000
_E = 3200000
_H = 10
_D = 20
_DP = 24


_BN = 4096


def _stage1_body(zt_ref, al_ref, ar_ref, el_ref, er_ref):
    zt = zt_ref[...]
    dn = (((0,), (0,)), ((), ()))
    el_ref[...] = lax.dot_general(zt, al_ref[...], dn,
                                  preferred_element_type=jnp.float32)
    er_ref[...] = lax.dot_general(zt, ar_ref[...], dn,
                                  preferred_element_type=jnp.float32)


def _edge_features(Zt, Al, Ar):
    grid = (_N + _BN - 1) // _BN
    return pl.pallas_call(
        _stage1_body,
        grid=(grid,),
        in_specs=[
            pl.BlockSpec((_H * _D, _BN), lambda i: (0, i)),
            pl.BlockSpec((_H * _D, _DP), lambda i: (0, 0)),
            pl.BlockSpec((_H * _D, _DP), lambda i: (0, 0)),
        ],
        out_specs=[
            pl.BlockSpec((_BN, _DP), lambda i: (i, 0)),
            pl.BlockSpec((_BN, _DP), lambda i: (i, 0)),
        ],
        out_shape=[
            jax.ShapeDtypeStruct((_N, _DP), jnp.float32),
            jax.ShapeDtypeStruct((_N, _DP), jnp.float32),
        ],
    )(Zt, Al, Ar)


_NC = 2
_NS = 16
_NW = _NC * _NS

_CH = 128
_NCHUNK = _E // _CH
_SB = 8

_CNT_HI = 784
_N_HI = 21

_VEC = 16


def _sc_gather_add(el, er, row2d, col2d):
    mesh = plsc.VectorSubcoreMesh(core_axis_name="c", subcore_axis_name="s")

    @functools.partial(
        pl.kernel,
        out_type=jax.ShapeDtypeStruct((3, _NCHUNK, 8, _CH), jnp.float32),
        mesh=mesh,
        scratch_types=[
            pltpu.VMEM((_SB, _CH), jnp.int32),
            pltpu.VMEM((_SB, _CH), jnp.int32),
            pltpu.VMEM((_SB, _CH, _DP), jnp.float32),
            pltpu.VMEM((_SB, _CH, _DP), jnp.float32),
            pltpu.VMEM((_SB, _CH, 21), jnp.float32),
            pltpu.VMEM((3, _SB, 8, _CH), jnp.float32),
            pltpu.SemaphoreType.DMA,
            pltpu.SemaphoreType.DMA,
        ],
        compiler_params=pltpu.CompilerParams(
            use_tc_tiling_on_sc=False, needs_layout_passes=False),
    )
    def k(el_hbm, er_hbm, row_hbm, col_hbm, out_hbm, idx1, idx2, buf1, buf2,
          bufo, buft, sem1, sem2):
        wid = lax.axis_index("s") * _NC + lax.axis_index("c")
        start = wid * _CNT_HI - jnp.maximum(wid - _N_HI, 0) * _SB
        start = pl.multiple_of(start, 8)

        def add_all():
            def body1(g, _):
                lane = lax.iota(jnp.int32, _VEC)
                for u in range(4):
                    o = (g * 4 + u) * _VEC + lane
                    r = o // 21
                    d = o - r * 21
                    j = r >> 7
                    rr = r & (_CH - 1)
                    a = plsc.load_gather(buf1, [j, rr, d])
                    b = plsc.load_gather(buf2, [j, rr, d])
                    plsc.store_scatter(bufo, [j, rr, d], a + b)
                return 0
            lax.fori_loop(0, _SB * _CH * 21 // (_VEC * 4), body1, 0)

            def body2(g, _):
                lane = lax.iota(jnp.int32, _VEC)
                rg = g * _VEC + lane
                j = rg >> 7
                rr = rg & (_CH - 1)
                for d in range(_D):
                    dv = jnp.full((_VEC,), d, jnp.int32)
                    d3 = jnp.full((_VEC,), d // 8, jnp.int32)
                    sv = jnp.full((_VEC,), d % 8, jnp.int32)
                    v = plsc.load_gather(bufo, [j, rr, dv])
                    plsc.store_scatter(buft, [d3, j, sv, rr], v)
                return 0
            lax.fori_loop(0, _SB * _CH // _VEC, body2, 0)

        def sb_body(s, _):
            c0 = pl.multiple_of(start + s * _SB, 8)
            pltpu.sync_copy(row_hbm.at[pl.ds(c0, _SB), :], idx1)
            pltpu.sync_copy(col_hbm.at[pl.ds(c0, _SB), :], idx2)
            waits = []
            for j in range(_SB):
                waits.append(
                    pltpu.async_copy(el_hbm.at[idx1.at[j]], buf1.at[j], sem1))
                waits.append(
                    pltpu.async_copy(er_hbm.at[idx2.at[j]], buf2.at[j], sem2))
            for w in waits:
                w.wait()
            add_all()
            for i in range(3):
                pltpu.sync_copy(buft.at[i],
                                out_hbm.at[i, pl.ds(c0, _SB), :, :])
            return 0

        n_sb = jnp.where(wid < _N_HI, _CNT_HI // _SB, _CNT_HI // _SB - 1)
        lax.fori_loop(0, n_sb, sb_body, 0)

    return k(el, er, row2d, col2d)


def kernel(Z, row, col, a_l, a_r):
    Zt = Z.reshape(_N, _H * _D).T
    eyep = jnp.concatenate(
        [jnp.eye(_D, dtype=jnp.float32),
         jnp.zeros((_D, _DP - _D), jnp.float32)], axis=1)
    Al = (a_l[0][:, :, None] * eyep[None]).reshape(_H * _D, _DP)
    Ar = (a_r[0][:, :, None] * eyep[None]).reshape(_H * _D, _DP)
    el, er = _edge_features(Zt, Al, Ar)

    row2d = row.astype(jnp.int32).reshape(_NCHUNK, _CH)
    col2d = col.astype(jnp.int32).reshape(_NCHUNK, _CH)
    out4d = _sc_gather_add(el, er, row2d, col2d)
    return out4d.transpose(1, 3, 0, 2).reshape(_E, 3 * 8)[:, :_D]


# speedup vs baseline: 1.4582x; 1.4582x over previous
import functools

import jax
import jax.numpy as jnp
from jax import lax
from jax.experimental import pallas as pl
from jax.experimental.pallas import tpu as pltpu
from jax.experimental.pallas import tpu_sc as plsc

_N = 100000
_E = 3200000
_H = 10
_D = 20
_DP = 24


_BN = 4096


def _stage1_body(zt_ref, al_ref, ar_ref, el_ref, er_ref):
    zt = zt_ref[...]
    dn = (((0,), (0,)), ((), ()))
    el_ref[...] = lax.dot_general(zt, al_ref[...], dn,
                                  preferred_element_type=jnp.float32)
    er_ref[...] = lax.dot_general(zt, ar_ref[...], dn,
                                  preferred_element_type=jnp.float32)


def _edge_features(Zt, Al, Ar):
    grid = (_N + _BN - 1) // _BN
    return pl.pallas_call(
        _stage1_body,
        grid=(grid,),
        in_specs=[
            pl.BlockSpec((_H * _D, _BN), lambda i: (0, i)),
            pl.BlockSpec((_H * _D, _DP), lambda i: (0, 0)),
            pl.BlockSpec((_H * _D, _DP), lambda i: (0, 0)),
        ],
        out_specs=[
            pl.BlockSpec((_BN, _DP), lambda i: (i, 0)),
            pl.BlockSpec((_BN, _DP), lambda i: (i, 0)),
        ],
        out_shape=[
            jax.ShapeDtypeStruct((_N, _DP), jnp.float32),
            jax.ShapeDtypeStruct((_N, _DP), jnp.float32),
        ],
    )(Zt, Al, Ar)


_NC = 2
_NS = 16
_NW = _NC * _NS

_CH = 128
_NCHUNK = _E // _CH
_SB = 8

_CNT_HI = 784
_N_HI = 21

_VEC = 16


def _sc_gather_add(el, er, row2d, col2d):
    mesh = plsc.VectorSubcoreMesh(core_axis_name="c", subcore_axis_name="s")

    @functools.partial(
        pl.kernel,
        out_type=jax.ShapeDtypeStruct((3, _NCHUNK, 8, _CH), jnp.float32),
        mesh=mesh,
        scratch_types=[
            pltpu.VMEM((_SB, _CH), jnp.int32),
            pltpu.VMEM((_SB, _CH), jnp.int32),
            pltpu.VMEM((_SB, _CH, _DP), jnp.float32),
            pltpu.VMEM((_SB, _CH, _DP), jnp.float32),
            pltpu.VMEM((_SB, _CH, 21), jnp.float32),
            pltpu.VMEM((3, _SB, 8, _CH), jnp.float32),
            pltpu.SemaphoreType.DMA,
            pltpu.SemaphoreType.DMA,
        ],
        compiler_params=pltpu.CompilerParams(
            use_tc_tiling_on_sc=False, needs_layout_passes=False),
    )
    def k(el_hbm, er_hbm, row_hbm, col_hbm, out_hbm, idx1, idx2, buf1, buf2,
          bufo, buft, sem1, sem2):
        wid = lax.axis_index("s") * _NC + lax.axis_index("c")
        start = wid * _CNT_HI - jnp.maximum(wid - _N_HI, 0) * _SB
        start = pl.multiple_of(start, 8)

        def add_all():
            @plsc.parallel_loop(0, _SB * _CH * _D // (_VEC * 5), unroll=2)
            def body(g):
                lane = lax.iota(jnp.int32, _VEC)
                roff = g * 4
                for u in range(5):
                    o = u * _VEC + lane
                    r = o // _D + roff
                    j = r // _CH
                    rr = r - j * _CH
                    d = o % _D
                    d3 = d // 8
                    s = d - d3 * 8
                    a = plsc.load_gather(buf1, [j, rr, d])
                    b = plsc.load_gather(buf2, [j, rr, d])
                    plsc.store_scatter(buft, [d3, j, s, rr], a + b)

        def sb_body(s, _):
            c0 = pl.multiple_of(start + s * _SB, 8)
            pltpu.sync_copy(row_hbm.at[pl.ds(c0, _SB), :], idx1)
            pltpu.sync_copy(col_hbm.at[pl.ds(c0, _SB), :], idx2)
            waits = []
            for j in range(_SB):
                waits.append(
                    pltpu.async_copy(el_hbm.at[idx1.at[j]], buf1.at[j], sem1))
                waits.append(
                    pltpu.async_copy(er_hbm.at[idx2.at[j]], buf2.at[j], sem2))
            for w in waits:
                w.wait()
            add_all()
            for i in range(3):
                pltpu.sync_copy(buft.at[i],
                                out_hbm.at[i, pl.ds(c0, _SB), :, :])
            return 0

        n_sb = jnp.where(wid < _N_HI, _CNT_HI // _SB, _CNT_HI // _SB - 1)
        lax.fori_loop(0, n_sb, sb_body, 0)

    return k(el, er, row2d, col2d)


def kernel(Z, row, col, a_l, a_r):
    Zt = Z.reshape(_N, _H * _D).T
    eyep = jnp.concatenate(
        [jnp.eye(_D, dtype=jnp.float32),
         jnp.zeros((_D, _DP - _D), jnp.float32)], axis=1)
    Al = (a_l[0][:, :, None] * eyep[None]).reshape(_H * _D, _DP)
    Ar = (a_r[0][:, :, None] * eyep[None]).reshape(_H * _D, _DP)
    el, er = _edge_features(Zt, Al, Ar)

    row2d = row.astype(jnp.int32).reshape(_NCHUNK, _CH)
    col2d = col.astype(jnp.int32).reshape(_NCHUNK, _CH)
    out4d = _sc_gather_add(el, er, row2d, col2d)
    return out4d.transpose(1, 3, 0, 2).reshape(_E, 3 * 8)[:, :_D]
